# baseline (speedup 1.0000x reference)
import functools
import math

import jax
import jax.numpy as jnp
from jax import lax
from jax.experimental import pallas as pl
from jax.experimental.pallas import tpu as pltpu
from jax.experimental.pallas import tpu_sc as plsc

D_MODEL_ = 768
N_HEADS_ = 12
D_HEAD_ = 64
K_ = 32
N_ = 2048
QB_ = 1024
NQB_ = N_ // QB_

_NC = 2
_NS = 16
_NW = _NC * _NS
RB_ = 16
P_ = 2
HP_ = N_HEADS_ // P_


def _sortd(x):
    return plsc.sort_key_val(x, x, descending=True)[0]


def _merge16(a, b):
    rb = jnp.flip(b, 0)
    hi = jnp.maximum(a, rb)
    lo = jnp.minimum(a, rb)
    return _sortd(hi), _sortd(lo)


def _bitonic16_desc(x):
    for d in (8, 4, 2, 1):
        lane = lax.iota(jnp.int32, 16)
        p = x.at[lane ^ d].get(mode="promise_in_bounds")
        mask = (lane & d) == 0
        x = jnp.where(mask, jnp.maximum(x, p), jnp.minimum(x, p))
    return x


def _merge32(a, b):
    a1, a2 = a
    b1, b2 = b
    c1 = jnp.maximum(a1, jnp.flip(b2, 0))
    c2 = jnp.maximum(a2, jnp.flip(b1, 0))
    m1 = jnp.maximum(c1, c2)
    m2 = jnp.minimum(c1, c2)
    return _bitonic16_desc(m1), _bitonic16_desc(m2)


def _row_top32(sl):
    gs = []
    for g in range(16):
        ss = [_sortd(sl(g * 8 + j)) for j in range(8)]
        m16 = [_merge16(ss[0], ss[1]), _merge16(ss[2], ss[3]),
               _merge16(ss[4], ss[5]), _merge16(ss[6], ss[7])]
        m32 = [_merge32(m16[0], m16[1]), _merge32(m16[2], m16[3])]
        gs.append(_merge32(m32[0], m32[1]))
    while len(gs) > 1:
        gs = [_merge32(gs[i], gs[i + 1]) for i in range(0, len(gs), 2)]
    return gs[0]


def _make_topk_sc_kernel(rpw, nb):
    def _topk_sc_kernel(s_hbm, ts_hbm, buf0, buf1, obuf, sem0, sem1):
        cid = lax.axis_index("c")
        sid = lax.axis_index("s")
        wid = sid * _NC + cid
        row0 = wid * rpw
        bufs_all = (buf0, buf1)
        sems = (sem0, sem1)

        def in_copy(b, slot):
            return pltpu.make_async_copy(
                s_hbm.at[pl.ds(row0 + b * RB_, RB_), :],
                bufs_all[slot], sems[slot])

        in_copy(0, 0).start()
        in_copy(1, 1).start()

        def process(b, slot):
            in_copy(b, slot).wait()
            bufs = bufs_all[slot]

            def rbody(r, c):
                def sl(i):
                    return bufs[r, pl.ds(i * 16, 16)]

                t1, t2 = _row_top32(sl)
                obuf[r, pl.ds(0, 16)] = t1
                obuf[r, pl.ds(16, 16)] = t2
                return c

            lax.fori_loop(0, RB_, rbody, 0)
            pltpu.sync_copy(obuf,
                            ts_hbm.at[pl.ds(row0 + b * RB_, RB_), :])

        def outer(i, c):
            b0 = 2 * i
            process(b0, 0)

            @pl.when(b0 + 2 < nb)
            def _():
                in_copy(b0 + 2, 0).start()

            process(b0 + 1, 1)

            @pl.when(b0 + 3 < nb)
            def _():
                in_copy(b0 + 3, 1).start()

            return c

        lax.fori_loop(0, nb // 2, outer, 0)

    return _topk_sc_kernel


def _topk_vals_sc(scores):
    nh = scores.shape[0]
    rows = nh * N_
    rpw = rows // _NW
    nb = rpw // RB_
    mesh = plsc.VectorSubcoreMesh(core_axis_name="c", subcore_axis_name="s")
    f = pl.kernel(
        _make_topk_sc_kernel(rpw, nb),
        mesh=mesh,
        out_type=jax.ShapeDtypeStruct((rows, K_), jnp.float32),
        compiler_params=pltpu.CompilerParams(needs_layout_passes=False),
        scratch_types=[
            pltpu.VMEM((RB_, N_), jnp.float32),
            pltpu.VMEM((RB_, N_), jnp.float32),
            pltpu.VMEM((RB_, K_), jnp.float32),
            pltpu.SemaphoreType.DMA,
            pltpu.SemaphoreType.DMA,
        ],
    )
    return f(scores.reshape(rows, N_)).reshape(nh, N_, K_)


def _scores_kernel(x_ref, wq_ref, bq_ref, wk_ref, bk_ref, wv_ref, bv_ref,
                   s_ref, v_ref):
    qb = pl.program_id(1)
    x = x_ref[...]
    xq = x_ref[pl.ds(qb * QB_, QB_), :]
    q = jnp.dot(xq, wq_ref[0], preferred_element_type=jnp.float32) + bq_ref[0]
    k = jnp.dot(x, wk_ref[0], preferred_element_type=jnp.float32) + bk_ref[0]
    v = jnp.dot(x, wv_ref[0], preferred_element_type=jnp.float32) + bv_ref[0]
    v_ref[0, :, :] = v
    s = jax.lax.dot_general(q, k, (((1,), (1,)), ((), ())),
                            preferred_element_type=jnp.float32)
    s_ref[0, :, :] = s * (1.0 / math.sqrt(D_HEAD_))


def _context_kernel(s_ref, ts_ref, v_ref, wo_ref, out_ref, aw_ref):
    h = pl.program_id(1)
    ts = ts_ref[0, :, :]
    m = ts[:, 0:1]
    t = ts[:, K_ - 1:K_]
    w = jnp.exp(ts - m)
    z = jnp.sum(w, axis=-1, keepdims=True)
    aw_ref[0, :, :] = w / z
    s = s_ref[0, :, :]
    p = jnp.where(s >= t, jnp.exp(s - m), 0.0)
    ctx = jnp.dot(p, v_ref[0, :, :], preferred_element_type=jnp.float32) / z
    contrib = jnp.dot(ctx, wo_ref[0], preferred_element_type=jnp.float32)

    @pl.when(h == 0)
    def _():
        out_ref[...] = contrib

    @pl.when(h > 0)
    def _():
        out_ref[...] = out_ref[...] + contrib


def kernel(x, Wq, bq, Wk, bk, Wv, bv, Wo, bo):
    x2 = x[0]
    wq3 = Wq.reshape(D_MODEL_, N_HEADS_, D_HEAD_).transpose(1, 0, 2)
    wk3 = Wk.reshape(D_MODEL_, N_HEADS_, D_HEAD_).transpose(1, 0, 2)
    wv3 = Wv.reshape(D_MODEL_, N_HEADS_, D_HEAD_).transpose(1, 0, 2)
    wo3 = Wo.reshape(N_HEADS_, D_HEAD_, D_MODEL_)
    bq3 = bq.reshape(N_HEADS_, 1, D_HEAD_)
    bk3 = bk.reshape(N_HEADS_, 1, D_HEAD_)
    bv3 = bv.reshape(N_HEADS_, 1, D_HEAD_)
    bo2 = bo.reshape(1, D_MODEL_)

    def scores_call(wq, bqp, wk, bkp, wv, bvp):
        hp = wq.shape[0]
        return pl.pallas_call(
            _scores_kernel,
            grid=(hp, NQB_),
            in_specs=[
                pl.BlockSpec((N_, D_MODEL_), lambda h, qb: (0, 0)),
                pl.BlockSpec((1, D_MODEL_, D_HEAD_), lambda h, qb: (h, 0, 0)),
                pl.BlockSpec((1, 1, D_HEAD_), lambda h, qb: (h, 0, 0)),
                pl.BlockSpec((1, D_MODEL_, D_HEAD_), lambda h, qb: (h, 0, 0)),
                pl.BlockSpec((1, 1, D_HEAD_), lambda h, qb: (h, 0, 0)),
                pl.BlockSpec((1, D_MODEL_, D_HEAD_), lambda h, qb: (h, 0, 0)),
                pl.BlockSpec((1, 1, D_HEAD_), lambda h, qb: (h, 0, 0)),
            ],
            out_specs=[
                pl.BlockSpec((1, QB_, N_), lambda h, qb: (h, qb, 0)),
                pl.BlockSpec((1, N_, D_HEAD_), lambda h, qb: (h, 0, 0)),
            ],
            out_shape=[
                jax.ShapeDtypeStruct((hp, N_, N_), jnp.float32),
                jax.ShapeDtypeStruct((hp, N_, D_HEAD_), jnp.float32),
            ],
        )(x2, wq, bqp, wk, bkp, wv, bvp)

    def context_call(s_p, ts_p, v_p, wo_p):
        hp = s_p.shape[0]
        return pl.pallas_call(
            _context_kernel,
            grid=(NQB_, hp),
            in_specs=[
                pl.BlockSpec((1, QB_, N_), lambda qb, h: (h, qb, 0)),
                pl.BlockSpec((1, QB_, K_), lambda qb, h: (h, qb, 0)),
                pl.BlockSpec((1, N_, D_HEAD_), lambda qb, h: (h, 0, 0)),
                pl.BlockSpec((1, D_HEAD_, D_MODEL_), lambda qb, h: (h, 0, 0)),
            ],
            out_specs=[
                pl.BlockSpec((QB_, D_MODEL_), lambda qb, h: (qb, 0)),
                pl.BlockSpec((1, QB_, K_), lambda qb, h: (h, qb, 0)),
            ],
            out_shape=[
                jax.ShapeDtypeStruct((N_, D_MODEL_), jnp.float32),
                jax.ShapeDtypeStruct((hp, N_, K_), jnp.float32),
            ],
        )(s_p, ts_p, v_p, wo_p)

    parts = []
    for p in range(P_):
        hs = slice(p * HP_, (p + 1) * HP_)
        s_p, v_p = scores_call(wq3[hs], bq3[hs], wk3[hs], bk3[hs],
                               wv3[hs], bv3[hs])
        ts_p = _topk_vals_sc(s_p)
        parts.append((s_p, ts_p, v_p))

    outs, aws = [], []
    for p, (s_p, ts_p, v_p) in enumerate(parts):
        o_p, aw_p = context_call(s_p, ts_p, v_p,
                                 wo3[p * HP_:(p + 1) * HP_])
        outs.append(o_p)
        aws.append(aw_p)

    out = functools.reduce(lambda a_, b_: a_ + b_, outs) + bo2
    aw = jnp.concatenate(aws, axis=0)
    return out[None], aw[None]


# speedup vs baseline: 1.0049x; 1.0049x over previous
import functools
import math

import jax
import jax.numpy as jnp
from jax import lax
from jax.experimental import pallas as pl
from jax.experimental.pallas import tpu as pltpu
from jax.experimental.pallas import tpu_sc as plsc

D_MODEL_ = 768
N_HEADS_ = 12
D_HEAD_ = 64
K_ = 32
N_ = 2048
QB_ = 1024
NQB_ = N_ // QB_

_NC = 2
_NS = 16
_NW = _NC * _NS
RB_ = 16
P_ = 3
HP_ = N_HEADS_ // P_


def _sortd(x):
    return plsc.sort_key_val(x, x, descending=True)[0]


def _merge16(a, b):
    rb = jnp.flip(b, 0)
    hi = jnp.maximum(a, rb)
    lo = jnp.minimum(a, rb)
    return _sortd(hi), _sortd(lo)


def _bitonic16_desc(x):
    for d in (8, 4, 2, 1):
        lane = lax.iota(jnp.int32, 16)
        p = x.at[lane ^ d].get(mode="promise_in_bounds")
        mask = (lane & d) == 0
        x = jnp.where(mask, jnp.maximum(x, p), jnp.minimum(x, p))
    return x


def _merge32(a, b):
    a1, a2 = a
    b1, b2 = b
    c1 = jnp.maximum(a1, jnp.flip(b2, 0))
    c2 = jnp.maximum(a2, jnp.flip(b1, 0))
    m1 = jnp.maximum(c1, c2)
    m2 = jnp.minimum(c1, c2)
    return _bitonic16_desc(m1), _bitonic16_desc(m2)


def _row_top32(sl):
    gs = []
    for g in range(16):
        ss = [_sortd(sl(g * 8 + j)) for j in range(8)]
        m16 = [_merge16(ss[0], ss[1]), _merge16(ss[2], ss[3]),
               _merge16(ss[4], ss[5]), _merge16(ss[6], ss[7])]
        m32 = [_merge32(m16[0], m16[1]), _merge32(m16[2], m16[3])]
        gs.append(_merge32(m32[0], m32[1]))
    while len(gs) > 1:
        gs = [_merge32(gs[i], gs[i + 1]) for i in range(0, len(gs), 2)]
    return gs[0]


def _make_topk_sc_kernel(rpw, nb):
    def _topk_sc_kernel(s_hbm, ts_hbm, buf0, buf1, obuf, sem0, sem1):
        cid = lax.axis_index("c")
        sid = lax.axis_index("s")
        wid = sid * _NC + cid
        row0 = wid * rpw
        bufs_all = (buf0, buf1)
        sems = (sem0, sem1)

        def in_copy(b, slot):
            return pltpu.make_async_copy(
                s_hbm.at[pl.ds(row0 + b * RB_, RB_), :],
                bufs_all[slot], sems[slot])

        in_copy(0, 0).start()
        in_copy(1, 1).start()

        def process(b, slot):
            in_copy(b, slot).wait()
            bufs = bufs_all[slot]

            def rbody(r, c):
                def sl(i):
                    return bufs[r, pl.ds(i * 16, 16)]

                t1, t2 = _row_top32(sl)
                obuf[r, pl.ds(0, 16)] = t1
                obuf[r, pl.ds(16, 16)] = t2
                return c

            lax.fori_loop(0, RB_, rbody, 0)
            pltpu.sync_copy(obuf,
                            ts_hbm.at[pl.ds(row0 + b * RB_, RB_), :])

        def outer(i, c):
            b0 = 2 * i
            process(b0, 0)

            @pl.when(b0 + 2 < nb)
            def _():
                in_copy(b0 + 2, 0).start()

            process(b0 + 1, 1)

            @pl.when(b0 + 3 < nb)
            def _():
                in_copy(b0 + 3, 1).start()

            return c

        lax.fori_loop(0, nb // 2, outer, 0)

    return _topk_sc_kernel


def _topk_vals_sc(scores):
    nh = scores.shape[0]
    rows = nh * N_
    rpw = rows // _NW
    nb = rpw // RB_
    mesh = plsc.VectorSubcoreMesh(core_axis_name="c", subcore_axis_name="s")
    f = pl.kernel(
        _make_topk_sc_kernel(rpw, nb),
        mesh=mesh,
        out_type=jax.ShapeDtypeStruct((rows, K_), jnp.float32),
        compiler_params=pltpu.CompilerParams(needs_layout_passes=False),
        scratch_types=[
            pltpu.VMEM((RB_, N_), jnp.float32),
            pltpu.VMEM((RB_, N_), jnp.float32),
            pltpu.VMEM((RB_, K_), jnp.float32),
            pltpu.SemaphoreType.DMA,
            pltpu.SemaphoreType.DMA,
        ],
    )
    return f(scores.reshape(rows, N_)).reshape(nh, N_, K_)


def _scores_kernel(x_ref, wq_ref, bq_ref, wk_ref, bk_ref, wv_ref, bv_ref,
                   s_ref, v_ref):
    qb = pl.program_id(1)
    x = x_ref[...]
    xq = x_ref[pl.ds(qb * QB_, QB_), :]
    q = jnp.dot(xq, wq_ref[0], preferred_element_type=jnp.float32) + bq_ref[0]
    k = jnp.dot(x, wk_ref[0], preferred_element_type=jnp.float32) + bk_ref[0]
    v = jnp.dot(x, wv_ref[0], preferred_element_type=jnp.float32) + bv_ref[0]
    v_ref[0, :, :] = v
    s = jax.lax.dot_general(q, k, (((1,), (1,)), ((), ())),
                            preferred_element_type=jnp.float32)
    s_ref[0, :, :] = s * (1.0 / math.sqrt(D_HEAD_))


def _context_kernel(s_ref, ts_ref, v_ref, wo_ref, out_ref, aw_ref):
    h = pl.program_id(1)
    ts = ts_ref[0, :, :]
    m = ts[:, 0:1]
    t = ts[:, K_ - 1:K_]
    w = jnp.exp(ts - m)
    z = jnp.sum(w, axis=-1, keepdims=True)
    aw_ref[0, :, :] = w / z
    s = s_ref[0, :, :]
    p = jnp.where(s >= t, jnp.exp(s - m), 0.0)
    ctx = jnp.dot(p, v_ref[0, :, :], preferred_element_type=jnp.float32) / z
    contrib = jnp.dot(ctx, wo_ref[0], preferred_element_type=jnp.float32)

    @pl.when(h == 0)
    def _():
        out_ref[...] = contrib

    @pl.when(h > 0)
    def _():
        out_ref[...] = out_ref[...] + contrib


def kernel(x, Wq, bq, Wk, bk, Wv, bv, Wo, bo):
    x2 = x[0]
    wq3 = Wq.reshape(D_MODEL_, N_HEADS_, D_HEAD_).transpose(1, 0, 2)
    wk3 = Wk.reshape(D_MODEL_, N_HEADS_, D_HEAD_).transpose(1, 0, 2)
    wv3 = Wv.reshape(D_MODEL_, N_HEADS_, D_HEAD_).transpose(1, 0, 2)
    wo3 = Wo.reshape(N_HEADS_, D_HEAD_, D_MODEL_)
    bq3 = bq.reshape(N_HEADS_, 1, D_HEAD_)
    bk3 = bk.reshape(N_HEADS_, 1, D_HEAD_)
    bv3 = bv.reshape(N_HEADS_, 1, D_HEAD_)
    bo2 = bo.reshape(1, D_MODEL_)

    def scores_call(wq, bqp, wk, bkp, wv, bvp):
        hp = wq.shape[0]
        return pl.pallas_call(
            _scores_kernel,
            grid=(hp, NQB_),
            in_specs=[
                pl.BlockSpec((N_, D_MODEL_), lambda h, qb: (0, 0)),
                pl.BlockSpec((1, D_MODEL_, D_HEAD_), lambda h, qb: (h, 0, 0)),
                pl.BlockSpec((1, 1, D_HEAD_), lambda h, qb: (h, 0, 0)),
                pl.BlockSpec((1, D_MODEL_, D_HEAD_), lambda h, qb: (h, 0, 0)),
                pl.BlockSpec((1, 1, D_HEAD_), lambda h, qb: (h, 0, 0)),
                pl.BlockSpec((1, D_MODEL_, D_HEAD_), lambda h, qb: (h, 0, 0)),
                pl.BlockSpec((1, 1, D_HEAD_), lambda h, qb: (h, 0, 0)),
            ],
            out_specs=[
                pl.BlockSpec((1, QB_, N_), lambda h, qb: (h, qb, 0)),
                pl.BlockSpec((1, N_, D_HEAD_), lambda h, qb: (h, 0, 0)),
            ],
            out_shape=[
                jax.ShapeDtypeStruct((hp, N_, N_), jnp.float32),
                jax.ShapeDtypeStruct((hp, N_, D_HEAD_), jnp.float32),
            ],
        )(x2, wq, bqp, wk, bkp, wv, bvp)

    def context_call(s_p, ts_p, v_p, wo_p):
        hp = s_p.shape[0]
        return pl.pallas_call(
            _context_kernel,
            grid=(NQB_, hp),
            in_specs=[
                pl.BlockSpec((1, QB_, N_), lambda qb, h: (h, qb, 0)),
                pl.BlockSpec((1, QB_, K_), lambda qb, h: (h, qb, 0)),
                pl.BlockSpec((1, N_, D_HEAD_), lambda qb, h: (h, 0, 0)),
                pl.BlockSpec((1, D_HEAD_, D_MODEL_), lambda qb, h: (h, 0, 0)),
            ],
            out_specs=[
                pl.BlockSpec((QB_, D_MODEL_), lambda qb, h: (qb, 0)),
                pl.BlockSpec((1, QB_, K_), lambda qb, h: (h, qb, 0)),
            ],
            out_shape=[
                jax.ShapeDtypeStruct((N_, D_MODEL_), jnp.float32),
                jax.ShapeDtypeStruct((hp, N_, K_), jnp.float32),
            ],
        )(s_p, ts_p, v_p, wo_p)

    parts = []
    for p in range(P_):
        hs = slice(p * HP_, (p + 1) * HP_)
        s_p, v_p = scores_call(wq3[hs], bq3[hs], wk3[hs], bk3[hs],
                               wv3[hs], bv3[hs])
        ts_p = _topk_vals_sc(s_p)
        parts.append((s_p, ts_p, v_p))

    outs, aws = [], []
    for p, (s_p, ts_p, v_p) in enumerate(parts):
        o_p, aw_p = context_call(s_p, ts_p, v_p,
                                 wo3[p * HP_:(p + 1) * HP_])
        outs.append(o_p)
        aws.append(aw_p)

    out = functools.reduce(lambda a_, b_: a_ + b_, outs) + bo2
    aw = jnp.concatenate(aws, axis=0)
    return out[None], aw[None]


# speedup vs baseline: 1.0075x; 1.0026x over previous
import functools
import math

import jax
import jax.numpy as jnp
from jax import lax
from jax.experimental import pallas as pl
from jax.experimental.pallas import tpu as pltpu
from jax.experimental.pallas import tpu_sc as plsc

D_MODEL_ = 768
N_HEADS_ = 12
D_HEAD_ = 64
K_ = 32
N_ = 2048
QB_ = 1024
NQB_ = N_ // QB_

_NC = 2
_NS = 16
_NW = _NC * _NS
RB_ = 16
PARTS_ = (5, 5, 2)


def _sortd(x):
    return plsc.sort_key_val(x, x, descending=True)[0]


def _merge16(a, b):
    rb = jnp.flip(b, 0)
    hi = jnp.maximum(a, rb)
    lo = jnp.minimum(a, rb)
    return _sortd(hi), _sortd(lo)


def _bitonic16_desc(x):
    for d in (8, 4, 2, 1):
        lane = lax.iota(jnp.int32, 16)
        p = x.at[lane ^ d].get(mode="promise_in_bounds")
        mask = (lane & d) == 0
        x = jnp.where(mask, jnp.maximum(x, p), jnp.minimum(x, p))
    return x


def _merge32(a, b):
    a1, a2 = a
    b1, b2 = b
    c1 = jnp.maximum(a1, jnp.flip(b2, 0))
    c2 = jnp.maximum(a2, jnp.flip(b1, 0))
    m1 = jnp.maximum(c1, c2)
    m2 = jnp.minimum(c1, c2)
    return _bitonic16_desc(m1), _bitonic16_desc(m2)


def _row_top32(sl):
    gs = []
    for g in range(16):
        ss = [_sortd(sl(g * 8 + j)) for j in range(8)]
        m16 = [_merge16(ss[0], ss[1]), _merge16(ss[2], ss[3]),
               _merge16(ss[4], ss[5]), _merge16(ss[6], ss[7])]
        m32 = [_merge32(m16[0], m16[1]), _merge32(m16[2], m16[3])]
        gs.append(_merge32(m32[0], m32[1]))
    while len(gs) > 1:
        gs = [_merge32(gs[i], gs[i + 1]) for i in range(0, len(gs), 2)]
    return gs[0]


def _make_topk_sc_kernel(rpw, nb):
    def _topk_sc_kernel(s_hbm, ts_hbm, buf0, buf1, obuf, sem0, sem1):
        cid = lax.axis_index("c")
        sid = lax.axis_index("s")
        wid = sid * _NC + cid
        row0 = wid * rpw
        bufs_all = (buf0, buf1)
        sems = (sem0, sem1)

        def in_copy(b, slot):
            return pltpu.make_async_copy(
                s_hbm.at[pl.ds(row0 + b * RB_, RB_), :],
                bufs_all[slot], sems[slot])

        in_copy(0, 0).start()
        in_copy(1, 1).start()

        def process(b, slot):
            in_copy(b, slot).wait()
            bufs = bufs_all[slot]

            def rbody(r, c):
                def sl(i):
                    return bufs[r, pl.ds(i * 16, 16)]

                t1, t2 = _row_top32(sl)
                obuf[r, pl.ds(0, 16)] = t1
                obuf[r, pl.ds(16, 16)] = t2
                return c

            lax.fori_loop(0, RB_, rbody, 0)
            pltpu.sync_copy(obuf,
                            ts_hbm.at[pl.ds(row0 + b * RB_, RB_), :])

        def outer(i, c):
            b0 = 2 * i
            process(b0, 0)

            @pl.when(b0 + 2 < nb)
            def _():
                in_copy(b0 + 2, 0).start()

            process(b0 + 1, 1)

            @pl.when(b0 + 3 < nb)
            def _():
                in_copy(b0 + 3, 1).start()

            return c

        lax.fori_loop(0, nb // 2, outer, 0)

    return _topk_sc_kernel


def _topk_vals_sc(scores):
    nh = scores.shape[0]
    rows = nh * N_
    rpw = rows // _NW
    nb = rpw // RB_
    mesh = plsc.VectorSubcoreMesh(core_axis_name="c", subcore_axis_name="s")
    f = pl.kernel(
        _make_topk_sc_kernel(rpw, nb),
        mesh=mesh,
        out_type=jax.ShapeDtypeStruct((rows, K_), jnp.float32),
        compiler_params=pltpu.CompilerParams(needs_layout_passes=False),
        scratch_types=[
            pltpu.VMEM((RB_, N_), jnp.float32),
            pltpu.VMEM((RB_, N_), jnp.float32),
            pltpu.VMEM((RB_, K_), jnp.float32),
            pltpu.SemaphoreType.DMA,
            pltpu.SemaphoreType.DMA,
        ],
    )
    return f(scores.reshape(rows, N_)).reshape(nh, N_, K_)


def _scores_kernel(x_ref, wq_ref, bq_ref, wk_ref, bk_ref, wv_ref, bv_ref,
                   s_ref, v_ref):
    qb = pl.program_id(1)
    x = x_ref[...]
    xq = x_ref[pl.ds(qb * QB_, QB_), :]
    q = jnp.dot(xq, wq_ref[0], preferred_element_type=jnp.float32) + bq_ref[0]
    k = jnp.dot(x, wk_ref[0], preferred_element_type=jnp.float32) + bk_ref[0]
    v = jnp.dot(x, wv_ref[0], preferred_element_type=jnp.float32) + bv_ref[0]
    v_ref[0, :, :] = v
    s = jax.lax.dot_general(q, k, (((1,), (1,)), ((), ())),
                            preferred_element_type=jnp.float32)
    s_ref[0, :, :] = s * (1.0 / math.sqrt(D_HEAD_))


def _context_kernel(s_ref, ts_ref, v_ref, wo_ref, out_ref, aw_ref):
    h = pl.program_id(1)
    ts = ts_ref[0, :, :]
    m = ts[:, 0:1]
    t = ts[:, K_ - 1:K_]
    w = jnp.exp(ts - m)
    z = jnp.sum(w, axis=-1, keepdims=True)
    aw_ref[0, :, :] = w / z
    s = s_ref[0, :, :]
    p = jnp.where(s >= t, jnp.exp(s - m), 0.0)
    ctx = jnp.dot(p, v_ref[0, :, :], preferred_element_type=jnp.float32) / z
    contrib = jnp.dot(ctx, wo_ref[0], preferred_element_type=jnp.float32)

    @pl.when(h == 0)
    def _():
        out_ref[...] = contrib

    @pl.when(h > 0)
    def _():
        out_ref[...] = out_ref[...] + contrib


def kernel(x, Wq, bq, Wk, bk, Wv, bv, Wo, bo):
    x2 = x[0]
    wq3 = Wq.reshape(D_MODEL_, N_HEADS_, D_HEAD_).transpose(1, 0, 2)
    wk3 = Wk.reshape(D_MODEL_, N_HEADS_, D_HEAD_).transpose(1, 0, 2)
    wv3 = Wv.reshape(D_MODEL_, N_HEADS_, D_HEAD_).transpose(1, 0, 2)
    wo3 = Wo.reshape(N_HEADS_, D_HEAD_, D_MODEL_)
    bq3 = bq.reshape(N_HEADS_, 1, D_HEAD_)
    bk3 = bk.reshape(N_HEADS_, 1, D_HEAD_)
    bv3 = bv.reshape(N_HEADS_, 1, D_HEAD_)
    bo2 = bo.reshape(1, D_MODEL_)

    def scores_call(wq, bqp, wk, bkp, wv, bvp):
        hp = wq.shape[0]
        return pl.pallas_call(
            _scores_kernel,
            grid=(hp, NQB_),
            in_specs=[
                pl.BlockSpec((N_, D_MODEL_), lambda h, qb: (0, 0)),
                pl.BlockSpec((1, D_MODEL_, D_HEAD_), lambda h, qb: (h, 0, 0)),
                pl.BlockSpec((1, 1, D_HEAD_), lambda h, qb: (h, 0, 0)),
                pl.BlockSpec((1, D_MODEL_, D_HEAD_), lambda h, qb: (h, 0, 0)),
                pl.BlockSpec((1, 1, D_HEAD_), lambda h, qb: (h, 0, 0)),
                pl.BlockSpec((1, D_MODEL_, D_HEAD_), lambda h, qb: (h, 0, 0)),
                pl.BlockSpec((1, 1, D_HEAD_), lambda h, qb: (h, 0, 0)),
            ],
            out_specs=[
                pl.BlockSpec((1, QB_, N_), lambda h, qb: (h, qb, 0)),
                pl.BlockSpec((1, N_, D_HEAD_), lambda h, qb: (h, 0, 0)),
            ],
            out_shape=[
                jax.ShapeDtypeStruct((hp, N_, N_), jnp.float32),
                jax.ShapeDtypeStruct((hp, N_, D_HEAD_), jnp.float32),
            ],
        )(x2, wq, bqp, wk, bkp, wv, bvp)

    def context_call(s_p, ts_p, v_p, wo_p):
        hp = s_p.shape[0]
        return pl.pallas_call(
            _context_kernel,
            grid=(NQB_, hp),
            in_specs=[
                pl.BlockSpec((1, QB_, N_), lambda qb, h: (h, qb, 0)),
                pl.BlockSpec((1, QB_, K_), lambda qb, h: (h, qb, 0)),
                pl.BlockSpec((1, N_, D_HEAD_), lambda qb, h: (h, 0, 0)),
                pl.BlockSpec((1, D_HEAD_, D_MODEL_), lambda qb, h: (h, 0, 0)),
            ],
            out_specs=[
                pl.BlockSpec((QB_, D_MODEL_), lambda qb, h: (qb, 0)),
                pl.BlockSpec((1, QB_, K_), lambda qb, h: (h, qb, 0)),
            ],
            out_shape=[
                jax.ShapeDtypeStruct((N_, D_MODEL_), jnp.float32),
                jax.ShapeDtypeStruct((hp, N_, K_), jnp.float32),
            ],
        )(s_p, ts_p, v_p, wo_p)

    bounds = []
    h0 = 0
    for hp in PARTS_:
        bounds.append((h0, h0 + hp))
        h0 += hp

    parts = []
    for lo, hi in bounds:
        hs = slice(lo, hi)
        s_p, v_p = scores_call(wq3[hs], bq3[hs], wk3[hs], bk3[hs],
                               wv3[hs], bv3[hs])
        ts_p = _topk_vals_sc(s_p)
        parts.append((s_p, ts_p, v_p))

    outs, aws = [], []
    for (lo, hi), (s_p, ts_p, v_p) in zip(bounds, parts):
        o_p, aw_p = context_call(s_p, ts_p, v_p, wo3[lo:hi])
        outs.append(o_p)
        aws.append(aw_p)

    out = functools.reduce(lambda a_, b_: a_ + b_, outs) + bo2
    aw = jnp.concatenate(aws, axis=0)
    return out[None], aw[None]
